# baseline (speedup 1.0000x reference)
import functools
import math

import jax
import jax.numpy as jnp
from jax import lax
from jax.experimental import pallas as pl
from jax.experimental.pallas import tpu as pltpu
from jax.experimental.pallas import tpu_sc as plsc

_SQ2 = math.sqrt(2.0)
_PI4 = 4.0 * math.pi
_N00 = math.sqrt(1.0 / _PI4)
_N10 = math.sqrt(3.0 / _PI4)
_N11 = math.sqrt(3.0 / _PI4 / 2.0)
_N20 = math.sqrt(5.0 / _PI4)
_N21 = math.sqrt(5.0 / _PI4 / 6.0)
_N22 = math.sqrt(5.0 / _PI4 / 24.0)
_N30 = math.sqrt(7.0 / _PI4)
_N31 = math.sqrt(7.0 / _PI4 / 12.0)
_N32 = math.sqrt(7.0 / _PI4 / 120.0)
_N33 = math.sqrt(7.0 / _PI4 / 720.0)

_LANES = 16
_IDXW = 128
_CHUNK = 2048


def _rsqrt_sc(v):
    bits = plsc.bitcast(v, jnp.int32)
    seed = plsc.bitcast(0x5F3759DF - lax.shift_right_logical(bits, 1),
                        jnp.float32)
    h = 0.5 * v
    for _ in range(2):
        seed = seed * (1.5 - h * seed * seed)
    return seed


def _basis16(dx, dy, dz):
    s = jnp.maximum(dx * dx + dy * dy + dz * dz, 1e-30)
    rinv = _rsqrt_sc(s)
    ct = jnp.minimum(jnp.maximum(dz * rinv, -1.0), 1.0)
    v = jnp.maximum(1.0 - ct * ct, 1e-14)
    st = v * _rsqrt_sc(v)
    rhoinv = _rsqrt_sc(jnp.maximum(dx * dx + dy * dy, 1e-30))
    cp = dx * rhoinv
    sp = dy * rhoinv
    c2 = cp * cp - sp * sp
    s2 = 2.0 * cp * sp
    c3 = cp * c2 - sp * s2
    s3 = sp * c2 + cp * s2
    st2 = st * st
    p21 = 3.0 * ct * st
    p22 = 3.0 * st2
    p31 = 1.5 * st * (5.0 * ct * ct - 1.0)
    p32 = 15.0 * ct * st2
    p33 = 15.0 * st2 * st
    return (
        jnp.full_like(ct, _N00),
        (-_SQ2 * _N11) * st * sp,
        _N10 * ct,
        (-_SQ2 * _N11) * st * cp,
        (_SQ2 * _N22) * p22 * s2,
        (-_SQ2 * _N21) * p21 * sp,
        _N20 * (1.5 * ct * ct - 0.5),
        (-_SQ2 * _N21) * p21 * cp,
        (_SQ2 * _N22) * p22 * c2,
        (-_SQ2 * _N33) * p33 * s3,
        (_SQ2 * _N32) * p32 * s2,
        (-_SQ2 * _N31) * p31 * sp,
        _N30 * ((2.5 * ct * ct - 1.5) * ct),
        (-_SQ2 * _N31) * p31 * cp,
        (_SQ2 * _N32) * p32 * c2,
        (-_SQ2 * _N33) * p33 * c3,
    )


def _fused_sc(table, packed, res_y, res_x):
    nrows, d = table.shape
    b = packed.shape[1]
    info = plsc.get_sparse_core_info()
    nc, ns = info.num_cores, info.num_subcores
    nw = nc * ns
    b_per_w = b // nw
    assert b % (nw * _CHUNK) == 0
    nchunk = b_per_w // _CHUNK
    nsub = _CHUNK // _IDXW
    ngrp = _CHUNK // _LANES

    mesh = plsc.VectorSubcoreMesh(core_axis_name="c", subcore_axis_name="s")

    @functools.partial(
        pl.kernel,
        mesh=mesh,
        compiler_params=pltpu.CompilerParams(
            use_tc_tiling_on_sc=False, needs_layout_passes=False),
        out_type=jax.ShapeDtypeStruct((3 * b,), jnp.float32),
        scratch_types=[
            pltpu.VMEM((5, _CHUNK), jnp.float32),
            pltpu.VMEM((nsub, _IDXW), jnp.int32),
            pltpu.VMEM((_CHUNK, d), jnp.float32),
            pltpu.VMEM((3 * _CHUNK,), jnp.float32),
            pltpu.SemaphoreType.DMA,
        ],
    )
    def fused_k(table_hbm, in_hbm, out_hbm, in_v, idx_v, rows_v, out_v, sem):
        wid = lax.axis_index("s") * nc + lax.axis_index("c")
        base = wid * b_per_w
        ymax = float(res_y - 1)
        xmax = float(res_x - 1)
        lane = lax.iota(jnp.int32, _LANES)
        for c in range(nchunk):
            off = base + c * _CHUNK
            pltpu.sync_copy(in_hbm.at[:, pl.ds(off, _CHUNK)], in_v)

            for k in range(nsub):
                def idx_grp(g, carry, k=k):
                    s0 = k * _IDXW + g * _LANES
                    yv = in_v[0, pl.ds(s0, _LANES)]
                    xv = in_v[1, pl.ds(s0, _LANES)]
                    yc = jnp.minimum(jnp.maximum(yv, 0.0), ymax)
                    xc = jnp.minimum(jnp.maximum(xv, 0.0), xmax)
                    idx_v[k, pl.ds(g * _LANES, _LANES)] = (
                        yc * float(res_x) + xc).astype(jnp.int32)
                    return carry
                lax.fori_loop(0, _IDXW // _LANES, idx_grp, 0)

            copies = [
                pltpu.async_copy(
                    table_hbm.at[idx_v.at[k]],
                    rows_v.at[pl.ds(k * _IDXW, _IDXW)],
                    sem,
                )
                for k in range(nsub)
            ]
            for cp_ in copies:
                cp_.wait()

            def grp(g, carry):
                s0 = g * _LANES
                dx = in_v[2, pl.ds(s0, _LANES)]
                dy = in_v[3, pl.ds(s0, _LANES)]
                dz = in_v[4, pl.ds(s0, _LANES)]
                cols = _basis16(dx, dy, dz)
                row_ids = s0 + lane
                acc = [None, None, None]
                for i in range(16):
                    w = cols[i]
                    for j in range(3):
                        col_ids = jnp.full((_LANES,), 3 * i + j, jnp.int32)
                        cc = plsc.load_gather(rows_v, [row_ids, col_ids])
                        acc[j] = w * cc if acc[j] is None else acc[j] + w * cc
                oix = (s0 + lane) * 3
                for j in range(3):
                    val = jnp.minimum(jnp.maximum(acc[j], 0.0), 1.0)
                    plsc.store_scatter(out_v, [oix + j], val)
                return carry
            lax.fori_loop(0, ngrp, grp, 0)

            pltpu.sync_copy(out_v, out_hbm.at[pl.ds(off * 3, 3 * _CHUNK)])

    return fused_k(table, packed)


def kernel(y, x, ray_dir, sh_data):
    res_y, res_x, nco, nch = sh_data.shape
    d = nco * nch
    b = y.shape[0]
    table = sh_data.reshape(res_y * res_x, d)
    packed = jnp.stack(
        [y, x, ray_dir[:, 0], ray_dir[:, 1], ray_dir[:, 2]], axis=0)
    out = _fused_sc(table, packed, res_y, res_x)
    return out.reshape(b, 3)


# speedup vs baseline: 1.0762x; 1.0762x over previous
import functools
import math

import jax
import jax.numpy as jnp
from jax import lax
from jax.experimental import pallas as pl
from jax.experimental.pallas import tpu as pltpu
from jax.experimental.pallas import tpu_sc as plsc

_SQ2 = math.sqrt(2.0)
_PI4 = 4.0 * math.pi
_N00 = math.sqrt(1.0 / _PI4)
_N10 = math.sqrt(3.0 / _PI4)
_N11 = math.sqrt(3.0 / _PI4 / 2.0)
_N20 = math.sqrt(5.0 / _PI4)
_N21 = math.sqrt(5.0 / _PI4 / 6.0)
_N22 = math.sqrt(5.0 / _PI4 / 24.0)
_N30 = math.sqrt(7.0 / _PI4)
_N31 = math.sqrt(7.0 / _PI4 / 12.0)
_N32 = math.sqrt(7.0 / _PI4 / 120.0)
_N33 = math.sqrt(7.0 / _PI4 / 720.0)

_LANES = 16
_IDXW = 128
_CHUNK = 2048


def _rsqrt_sc(v):
    bits = plsc.bitcast(v, jnp.int32)
    seed = plsc.bitcast(0x5F3759DF - lax.shift_right_logical(bits, 1),
                        jnp.float32)
    h = 0.5 * v
    for _ in range(2):
        seed = seed * (1.5 - h * seed * seed)
    return seed


def _basis16(dx, dy, dz):
    s = jnp.maximum(dx * dx + dy * dy + dz * dz, 1e-30)
    rinv = _rsqrt_sc(s)
    ct = jnp.minimum(jnp.maximum(dz * rinv, -1.0), 1.0)
    v = jnp.maximum(1.0 - ct * ct, 1e-14)
    st = v * _rsqrt_sc(v)
    rhoinv = _rsqrt_sc(jnp.maximum(dx * dx + dy * dy, 1e-30))
    cp = dx * rhoinv
    sp = dy * rhoinv
    c2 = cp * cp - sp * sp
    s2 = 2.0 * cp * sp
    c3 = cp * c2 - sp * s2
    s3 = sp * c2 + cp * s2
    st2 = st * st
    p21 = 3.0 * ct * st
    p22 = 3.0 * st2
    p31 = 1.5 * st * (5.0 * ct * ct - 1.0)
    p32 = 15.0 * ct * st2
    p33 = 15.0 * st2 * st
    return (
        jnp.full_like(ct, _N00),
        (-_SQ2 * _N11) * st * sp,
        _N10 * ct,
        (-_SQ2 * _N11) * st * cp,
        (_SQ2 * _N22) * p22 * s2,
        (-_SQ2 * _N21) * p21 * sp,
        _N20 * (1.5 * ct * ct - 0.5),
        (-_SQ2 * _N21) * p21 * cp,
        (_SQ2 * _N22) * p22 * c2,
        (-_SQ2 * _N33) * p33 * s3,
        (_SQ2 * _N32) * p32 * s2,
        (-_SQ2 * _N31) * p31 * sp,
        _N30 * ((2.5 * ct * ct - 1.5) * ct),
        (-_SQ2 * _N31) * p31 * cp,
        (_SQ2 * _N32) * p32 * c2,
        (-_SQ2 * _N33) * p33 * c3,
    )


def _fused_sc(table, yf, xf, dxf, dyf, dzf, res_y, res_x):
    nrows, d = table.shape
    b = yf.shape[0]
    info = plsc.get_sparse_core_info()
    nc, ns = info.num_cores, info.num_subcores
    nw = nc * ns
    b_per_w = b // nw
    assert b % (nw * _CHUNK) == 0
    nchunk = b_per_w // _CHUNK
    nsub = _CHUNK // _IDXW
    ngrp = _CHUNK // _LANES

    mesh = plsc.VectorSubcoreMesh(core_axis_name="c", subcore_axis_name="s")

    @functools.partial(
        pl.kernel,
        mesh=mesh,
        compiler_params=pltpu.CompilerParams(
            use_tc_tiling_on_sc=False, needs_layout_passes=False),
        out_type=jax.ShapeDtypeStruct((3 * b,), jnp.float32),
        scratch_types=[
            pltpu.VMEM((_CHUNK,), jnp.float32),
            pltpu.VMEM((_CHUNK,), jnp.float32),
            pltpu.VMEM((_CHUNK,), jnp.float32),
            pltpu.VMEM((_CHUNK,), jnp.float32),
            pltpu.VMEM((_CHUNK,), jnp.float32),
            pltpu.VMEM((nsub, _IDXW), jnp.int32),
            pltpu.VMEM((_CHUNK, d), jnp.float32),
            pltpu.VMEM((3 * _CHUNK,), jnp.float32),
            pltpu.SemaphoreType.DMA,
            pltpu.SemaphoreType.DMA,
        ],
    )
    def fused_k(table_hbm, y_hbm, x_hbm, dx_hbm, dy_hbm, dz_hbm, out_hbm,
                y_v, x_v, dx_v, dy_v, dz_v, idx_v, rows_v, out_v,
                sem, sem_in):
        wid = lax.axis_index("s") * nc + lax.axis_index("c")
        base = wid * b_per_w
        ymax = float(res_y - 1)
        xmax = float(res_x - 1)
        lane = lax.iota(jnp.int32, _LANES)

        def chunk_body(c, carry):
            off = base + c * _CHUNK
            in_copies = [
                pltpu.async_copy(h.at[pl.ds(off, _CHUNK)], v, sem_in)
                for h, v in ((y_hbm, y_v), (x_hbm, x_v), (dx_hbm, dx_v),
                             (dy_hbm, dy_v), (dz_hbm, dz_v))
            ]
            for cp_ in in_copies:
                cp_.wait()

            for k in range(nsub):
                @plsc.parallel_loop(0, _IDXW // _LANES, 1, unroll=2)
                def idx_grp(g, k=k):
                    s0 = k * _IDXW + g * _LANES
                    yv = y_v[pl.ds(s0, _LANES)]
                    xv = x_v[pl.ds(s0, _LANES)]
                    yc = jnp.minimum(jnp.maximum(yv, 0.0), ymax)
                    xc = jnp.minimum(jnp.maximum(xv, 0.0), xmax)
                    idx_v[k, pl.ds(g * _LANES, _LANES)] = (
                        yc * float(res_x) + xc).astype(jnp.int32)

            copies = [
                pltpu.async_copy(
                    table_hbm.at[idx_v.at[k]],
                    rows_v.at[pl.ds(k * _IDXW, _IDXW)],
                    sem,
                )
                for k in range(nsub)
            ]
            for cp_ in copies:
                cp_.wait()

            @plsc.parallel_loop(0, ngrp, 1, unroll=2)
            def grp(g):
                s0 = g * _LANES
                dx = dx_v[pl.ds(s0, _LANES)]
                dy = dy_v[pl.ds(s0, _LANES)]
                dz = dz_v[pl.ds(s0, _LANES)]
                cols = _basis16(dx, dy, dz)
                row_ids = s0 + lane
                acc_a = [None, None, None]
                acc_b = [None, None, None]
                for i in range(16):
                    w = cols[i]
                    acc = acc_a if i < 8 else acc_b
                    for j in range(3):
                        col_ids = jnp.full((_LANES,), 3 * i + j, jnp.int32)
                        cc = plsc.load_gather(rows_v, [row_ids, col_ids])
                        acc[j] = w * cc if acc[j] is None else acc[j] + w * cc
                oix = (s0 + lane) * 3
                for j in range(3):
                    val = jnp.minimum(
                        jnp.maximum(acc_a[j] + acc_b[j], 0.0), 1.0)
                    plsc.store_scatter(out_v, [oix + j], val)

            pltpu.sync_copy(out_v, out_hbm.at[pl.ds(off * 3, 3 * _CHUNK)])
            return carry

        lax.fori_loop(0, nchunk, chunk_body, 0)

    return fused_k(table, yf, xf, dxf, dyf, dzf)


def kernel(y, x, ray_dir, sh_data):
    res_y, res_x, nco, nch = sh_data.shape
    d = nco * nch
    b = y.shape[0]
    table = sh_data.reshape(res_y * res_x, d)
    out = _fused_sc(table, y, x, ray_dir[:, 0], ray_dir[:, 1], ray_dir[:, 2],
                    res_y, res_x)
    return out.reshape(b, 3)


# speedup vs baseline: 1.6856x; 1.5663x over previous
import functools
import math

import jax
import jax.numpy as jnp
from jax import lax
from jax.experimental import pallas as pl
from jax.experimental.pallas import tpu as pltpu
from jax.experimental.pallas import tpu_sc as plsc

_SQ2 = math.sqrt(2.0)
_PI4 = 4.0 * math.pi
_N00 = math.sqrt(1.0 / _PI4)
_N10 = math.sqrt(3.0 / _PI4)
_N11 = math.sqrt(3.0 / _PI4 / 2.0)
_N20 = math.sqrt(5.0 / _PI4)
_N21 = math.sqrt(5.0 / _PI4 / 6.0)
_N22 = math.sqrt(5.0 / _PI4 / 24.0)
_N30 = math.sqrt(7.0 / _PI4)
_N31 = math.sqrt(7.0 / _PI4 / 12.0)
_N32 = math.sqrt(7.0 / _PI4 / 120.0)
_N33 = math.sqrt(7.0 / _PI4 / 720.0)

_LANES = 16
_IDXW = 128
_CHUNK = 2048


def _rsqrt_sc(v):
    bits = plsc.bitcast(v, jnp.int32)
    seed = plsc.bitcast(0x5F3759DF - lax.shift_right_logical(bits, 1),
                        jnp.float32)
    h = 0.5 * v
    for _ in range(2):
        seed = seed * (1.5 - h * seed * seed)
    return seed


def _basis16(dx, dy, dz):
    s = jnp.maximum(dx * dx + dy * dy + dz * dz, 1e-30)
    rinv = _rsqrt_sc(s)
    ct = jnp.minimum(jnp.maximum(dz * rinv, -1.0), 1.0)
    v = jnp.maximum(1.0 - ct * ct, 1e-14)
    st = v * _rsqrt_sc(v)
    rhoinv = _rsqrt_sc(jnp.maximum(dx * dx + dy * dy, 1e-30))
    cp = dx * rhoinv
    sp = dy * rhoinv
    c2 = cp * cp - sp * sp
    s2 = 2.0 * cp * sp
    c3 = cp * c2 - sp * s2
    s3 = sp * c2 + cp * s2
    st2 = st * st
    p21 = 3.0 * ct * st
    p22 = 3.0 * st2
    p31 = 1.5 * st * (5.0 * ct * ct - 1.0)
    p32 = 15.0 * ct * st2
    p33 = 15.0 * st2 * st
    return (
        jnp.full_like(ct, _N00),
        (-_SQ2 * _N11) * st * sp,
        _N10 * ct,
        (-_SQ2 * _N11) * st * cp,
        (_SQ2 * _N22) * p22 * s2,
        (-_SQ2 * _N21) * p21 * sp,
        _N20 * (1.5 * ct * ct - 0.5),
        (-_SQ2 * _N21) * p21 * cp,
        (_SQ2 * _N22) * p22 * c2,
        (-_SQ2 * _N33) * p33 * s3,
        (_SQ2 * _N32) * p32 * s2,
        (-_SQ2 * _N31) * p31 * sp,
        _N30 * ((2.5 * ct * ct - 1.5) * ct),
        (-_SQ2 * _N31) * p31 * cp,
        (_SQ2 * _N32) * p32 * c2,
        (-_SQ2 * _N33) * p33 * c3,
    )


def _fused_sc(table, yf, xf, dxf, dyf, dzf, res_y, res_x):
    nrows, d = table.shape
    b = yf.shape[0]
    info = plsc.get_sparse_core_info()
    nc, ns = info.num_cores, info.num_subcores
    nw = nc * ns
    b_per_w = b // nw
    assert b % (nw * _CHUNK) == 0
    nchunk = b_per_w // _CHUNK
    nsub = _CHUNK // _IDXW
    ngrp = _CHUNK // _LANES

    mesh = plsc.VectorSubcoreMesh(core_axis_name="c", subcore_axis_name="s")

    @functools.partial(
        pl.kernel,
        mesh=mesh,
        compiler_params=pltpu.CompilerParams(
            use_tc_tiling_on_sc=False, needs_layout_passes=False),
        out_type=jax.ShapeDtypeStruct((3, b), jnp.float32),
        scratch_types=[
            pltpu.VMEM((_CHUNK,), jnp.float32),
            pltpu.VMEM((_CHUNK,), jnp.float32),
            pltpu.VMEM((_CHUNK,), jnp.float32),
            pltpu.VMEM((_CHUNK,), jnp.float32),
            pltpu.VMEM((_CHUNK,), jnp.float32),
            pltpu.VMEM((nsub, _IDXW), jnp.int32),
            pltpu.VMEM((_CHUNK, d), jnp.float32),
            pltpu.VMEM((3, _CHUNK), jnp.float32),
            pltpu.SemaphoreType.DMA,
            pltpu.SemaphoreType.DMA,
        ],
    )
    def fused_k(table_hbm, y_hbm, x_hbm, dx_hbm, dy_hbm, dz_hbm, out_hbm,
                y_v, x_v, dx_v, dy_v, dz_v, idx_v, rows_v, out_v,
                sem, sem_in):
        wid = lax.axis_index("s") * nc + lax.axis_index("c")
        base = wid * b_per_w
        ymax = float(res_y - 1)
        xmax = float(res_x - 1)
        lane = lax.iota(jnp.int32, _LANES)

        def chunk_body(c, carry):
            off = base + c * _CHUNK
            in_copies = [
                pltpu.async_copy(h.at[pl.ds(off, _CHUNK)], v, sem_in)
                for h, v in ((y_hbm, y_v), (x_hbm, x_v), (dx_hbm, dx_v),
                             (dy_hbm, dy_v), (dz_hbm, dz_v))
            ]
            for cp_ in in_copies:
                cp_.wait()

            for k in range(nsub):
                @plsc.parallel_loop(0, _IDXW // _LANES, 1, unroll=2)
                def idx_grp(g, k=k):
                    s0 = k * _IDXW + g * _LANES
                    yv = y_v[pl.ds(s0, _LANES)]
                    xv = x_v[pl.ds(s0, _LANES)]
                    yc = jnp.minimum(jnp.maximum(yv, 0.0), ymax)
                    xc = jnp.minimum(jnp.maximum(xv, 0.0), xmax)
                    idx_v[k, pl.ds(g * _LANES, _LANES)] = (
                        yc * float(res_x) + xc).astype(jnp.int32)

            copies = [
                pltpu.async_copy(
                    table_hbm.at[idx_v.at[k]],
                    rows_v.at[pl.ds(k * _IDXW, _IDXW)],
                    sem,
                )
                for k in range(nsub)
            ]
            for cp_ in copies:
                cp_.wait()

            @plsc.parallel_loop(0, ngrp, 1, unroll=2)
            def grp(g):
                s0 = g * _LANES
                dx = dx_v[pl.ds(s0, _LANES)]
                dy = dy_v[pl.ds(s0, _LANES)]
                dz = dz_v[pl.ds(s0, _LANES)]
                cols = _basis16(dx, dy, dz)
                row_ids = s0 + lane
                acc_a = [None, None, None]
                acc_b = [None, None, None]
                for i in range(16):
                    w = cols[i]
                    acc = acc_a if i < 8 else acc_b
                    for j in range(3):
                        col_ids = jnp.full((_LANES,), 3 * i + j, jnp.int32)
                        cc = plsc.load_gather(rows_v, [row_ids, col_ids])
                        acc[j] = w * cc if acc[j] is None else acc[j] + w * cc
                for j in range(3):
                    val = jnp.minimum(
                        jnp.maximum(acc_a[j] + acc_b[j], 0.0), 1.0)
                    out_v[j, pl.ds(s0, _LANES)] = val

            pltpu.sync_copy(out_v, out_hbm.at[:, pl.ds(off, _CHUNK)])
            return carry

        lax.fori_loop(0, nchunk, chunk_body, 0)

    return fused_k(table, yf, xf, dxf, dyf, dzf)


def kernel(y, x, ray_dir, sh_data):
    res_y, res_x, nco, nch = sh_data.shape
    d = nco * nch
    b = y.shape[0]
    table = sh_data.reshape(res_y * res_x, d)
    out = _fused_sc(table, y, x, ray_dir[:, 0], ray_dir[:, 1], ray_dir[:, 2],
                    res_y, res_x)
    return out.T


# speedup vs baseline: 1.7810x; 1.0565x over previous
import functools
import math

import jax
import jax.numpy as jnp
from jax import lax
from jax.experimental import pallas as pl
from jax.experimental.pallas import tpu as pltpu
from jax.experimental.pallas import tpu_sc as plsc

_SQ2 = math.sqrt(2.0)
_PI4 = 4.0 * math.pi
_N00 = math.sqrt(1.0 / _PI4)
_N10 = math.sqrt(3.0 / _PI4)
_N11 = math.sqrt(3.0 / _PI4 / 2.0)
_N20 = math.sqrt(5.0 / _PI4)
_N21 = math.sqrt(5.0 / _PI4 / 6.0)
_N22 = math.sqrt(5.0 / _PI4 / 24.0)
_N30 = math.sqrt(7.0 / _PI4)
_N31 = math.sqrt(7.0 / _PI4 / 12.0)
_N32 = math.sqrt(7.0 / _PI4 / 120.0)
_N33 = math.sqrt(7.0 / _PI4 / 720.0)

_LANES = 16
_IDXW = 128
_CHUNK = 1024


def _rsqrt_sc(v):
    bits = plsc.bitcast(v, jnp.int32)
    seed = plsc.bitcast(0x5F3759DF - lax.shift_right_logical(bits, 1),
                        jnp.float32)
    h = 0.5 * v
    for _ in range(2):
        seed = seed * (1.5 - h * seed * seed)
    return seed


def _basis16(dx, dy, dz):
    s = jnp.maximum(dx * dx + dy * dy + dz * dz, 1e-30)
    rinv = _rsqrt_sc(s)
    ct = jnp.minimum(jnp.maximum(dz * rinv, -1.0), 1.0)
    v = jnp.maximum(1.0 - ct * ct, 1e-14)
    st = v * _rsqrt_sc(v)
    rhoinv = _rsqrt_sc(jnp.maximum(dx * dx + dy * dy, 1e-30))
    cp = dx * rhoinv
    sp = dy * rhoinv
    c2 = cp * cp - sp * sp
    s2 = 2.0 * cp * sp
    c3 = cp * c2 - sp * s2
    s3 = sp * c2 + cp * s2
    st2 = st * st
    p21 = 3.0 * ct * st
    p22 = 3.0 * st2
    p31 = 1.5 * st * (5.0 * ct * ct - 1.0)
    p32 = 15.0 * ct * st2
    p33 = 15.0 * st2 * st
    return (
        jnp.full_like(ct, _N00),
        (-_SQ2 * _N11) * st * sp,
        _N10 * ct,
        (-_SQ2 * _N11) * st * cp,
        (_SQ2 * _N22) * p22 * s2,
        (-_SQ2 * _N21) * p21 * sp,
        _N20 * (1.5 * ct * ct - 0.5),
        (-_SQ2 * _N21) * p21 * cp,
        (_SQ2 * _N22) * p22 * c2,
        (-_SQ2 * _N33) * p33 * s3,
        (_SQ2 * _N32) * p32 * s2,
        (-_SQ2 * _N31) * p31 * sp,
        _N30 * ((2.5 * ct * ct - 1.5) * ct),
        (-_SQ2 * _N31) * p31 * cp,
        (_SQ2 * _N32) * p32 * c2,
        (-_SQ2 * _N33) * p33 * c3,
    )


def _fused_sc(table, yf, xf, dxf, dyf, dzf, res_y, res_x):
    nrows, d = table.shape
    b = yf.shape[0]
    info = plsc.get_sparse_core_info()
    nc, ns = info.num_cores, info.num_subcores
    nw = nc * ns
    b_per_w = b // nw
    assert b % (nw * 2 * _CHUNK) == 0
    nchunk = b_per_w // _CHUNK
    npair = nchunk // 2
    nsub = _CHUNK // _IDXW
    ngrp = _CHUNK // _LANES

    mesh = plsc.VectorSubcoreMesh(core_axis_name="c", subcore_axis_name="s")

    @functools.partial(
        pl.kernel,
        mesh=mesh,
        compiler_params=pltpu.CompilerParams(
            use_tc_tiling_on_sc=False, needs_layout_passes=False),
        out_type=jax.ShapeDtypeStruct((3, b), jnp.float32),
        scratch_types=[
            pltpu.VMEM((2, 5, _CHUNK), jnp.float32),
            pltpu.VMEM((2, nsub, _IDXW), jnp.int32),
            pltpu.VMEM((2 * _CHUNK, d), jnp.float32),
            pltpu.VMEM((3, _CHUNK), jnp.float32),
            pltpu.SemaphoreType.DMA,
            pltpu.SemaphoreType.DMA,
            pltpu.SemaphoreType.DMA,
        ],
    )
    def fused_k(table_hbm, y_hbm, x_hbm, dx_hbm, dy_hbm, dz_hbm, out_hbm,
                in_v, idx_v, rows_v, o_v, sem0, sem1, sem_in):
        wid = lax.axis_index("s") * nc + lax.axis_index("c")
        base = wid * b_per_w
        ymax = float(res_y - 1)
        xmax = float(res_x - 1)
        lane = lax.iota(jnp.int32, _LANES)
        sems = (sem0, sem1)

        def load_inputs(c, slot):
            off = base + c * _CHUNK
            cps = [
                pltpu.async_copy(h.at[pl.ds(off, _CHUNK)],
                                 in_v.at[slot, q], sem_in)
                for q, h in enumerate((y_hbm, x_hbm, dx_hbm, dy_hbm, dz_hbm))
            ]
            for cp_ in cps:
                cp_.wait()

        def compute_idx(slot):
            for k in range(nsub):
                @plsc.parallel_loop(0, _IDXW // _LANES, 1, unroll=2)
                def idx_grp(g, k=k):
                    s0 = k * _IDXW + g * _LANES
                    yv = in_v[slot, 0, pl.ds(s0, _LANES)]
                    xv = in_v[slot, 1, pl.ds(s0, _LANES)]
                    yc = jnp.minimum(jnp.maximum(yv, 0.0), ymax)
                    xc = jnp.minimum(jnp.maximum(xv, 0.0), xmax)
                    idx_v[slot, k, pl.ds(g * _LANES, _LANES)] = (
                        yc * float(res_x) + xc).astype(jnp.int32)

        def gather_copies(slot):
            return [
                pltpu.make_async_copy(
                    table_hbm.at[idx_v.at[slot, k]],
                    rows_v.at[pl.ds((slot * nsub + k) * _IDXW, _IDXW)],
                    sems[slot],
                )
                for k in range(nsub)
            ]

        def fire_gather(slot):
            for cp_ in gather_copies(slot):
                cp_.start()

        def drain_gather(slot):
            for cp_ in gather_copies(slot):
                cp_.wait()

        def prep_chunk(c, slot):
            load_inputs(c, slot)
            compute_idx(slot)
            fire_gather(slot)

        def compute_chunk(c, slot):
            off = base + c * _CHUNK

            @plsc.parallel_loop(0, ngrp, 1, unroll=2)
            def grp(g):
                s0 = g * _LANES
                dx = in_v[slot, 2, pl.ds(s0, _LANES)]
                dy = in_v[slot, 3, pl.ds(s0, _LANES)]
                dz = in_v[slot, 4, pl.ds(s0, _LANES)]
                cols = _basis16(dx, dy, dz)
                row_ids = slot * _CHUNK + s0 + lane
                acc_a = [None, None, None]
                acc_b = [None, None, None]
                for i in range(16):
                    w = cols[i]
                    acc = acc_a if i < 8 else acc_b
                    for j in range(3):
                        col_ids = jnp.full((_LANES,), 3 * i + j, jnp.int32)
                        cc = plsc.load_gather(rows_v, [row_ids, col_ids])
                        acc[j] = w * cc if acc[j] is None else acc[j] + w * cc
                for j in range(3):
                    val = jnp.minimum(
                        jnp.maximum(acc_a[j] + acc_b[j], 0.0), 1.0)
                    o_v[j, pl.ds(s0, _LANES)] = val

            for j in range(3):
                pltpu.sync_copy(o_v.at[j], out_hbm.at[j, pl.ds(off, _CHUNK)])

        prep_chunk(0, 0)

        def pair_body(p, carry):
            c0 = 2 * p
            c1 = c0 + 1
            prep_chunk(c1, 1)
            drain_gather(0)
            compute_chunk(c0, 0)

            @pl.when(p < npair - 1)
            def _():
                prep_chunk(c0 + 2, 0)

            drain_gather(1)
            compute_chunk(c1, 1)
            return carry

        lax.fori_loop(0, npair, pair_body, 0)

    return fused_k(table, yf, xf, dxf, dyf, dzf)


def kernel(y, x, ray_dir, sh_data):
    res_y, res_x, nco, nch = sh_data.shape
    d = nco * nch
    table = sh_data.reshape(res_y * res_x, d)
    out = _fused_sc(table, y, x, ray_dir[:, 0], ray_dir[:, 1], ray_dir[:, 2],
                    res_y, res_x)
    return out.T
